# Initial kernel scaffold; baseline (speedup 1.0000x reference)
import jax
import jax.numpy as jnp
from jax.experimental import pallas as pl


def kernel(dist, stops, weekday, vehicles, markov, demand, capacity, mask, W1, as1, ad1, We1, ae1, b1, W2, as2, ad2, We2, ae2, b2, Wes, bes, wemb, cemb, vemb, Wc1, bc1, Wc2, bc2):
    raise NotImplementedError("write your pallas kernel here")


# speedup vs baseline: 356.5838x; 356.5838x over previous
import jax
import jax.numpy as jnp
from jax.experimental import pallas as pl
from jax.experimental.pallas import tpu as pltpu

N = 256
H = 8
C = 32
ER = 32
FE = 3
HC = H * C

_NEG = -1e30


def _body(idx_ref, dist_ref, markov_ref, mask_ref, dem_ref, demt_ref,
          stops_ref, W1_ref, as1_ref, ad1_ref, We1_ref, ae1_ref, b1_ref,
          W2_ref, as2_ref, ad2_ref, We2_ref, ae2_ref, b2_ref,
          wes_top_ref, wes_bot_ref, bes_ref,
          wemb_ref, cemb_ref, vemb_ref,
          w3_ref, wc1_d_ref, wc1_m_ref, wc1_w_ref, wc1_c_ref, wc1_v_ref,
          wc1_sf_ref, bc1_ref, wc2_ref, bc2_ref, out_ref):
    f32 = jnp.float32

    dd = dist_ref[:, :]
    dn = (dd - jnp.min(dd)) / (jnp.max(dd) - jnp.min(dd))
    mk = markov_ref[:, :]
    valid = mask_ref[:, :] == 1
    dem = dem_ref[:, :]
    demt = demt_ref[:, :]

    def col_softmax(al):
        al = jnp.where(al >= 0.0, al, 0.2 * al)
        al = jnp.where(valid, al, _NEG)
        amax = jnp.max(al, axis=0, keepdims=True)
        amax = jnp.where(amax > -1e29, amax, 0.0)
        ex = jnp.exp(al - amax)
        den = jnp.sum(ex, axis=0, keepdims=True)
        return ex / (den + 1e-16)

    x1_blocks = []
    for h in range(H):
        sl = slice(h * C, (h + 1) * C)
        W1h = W1_ref[:, sl]
        p1 = jnp.sum(W1h * as1_ref[h:h + 1, :])
        q1 = jnp.sum(W1h * ad1_ref[h:h + 1, :])
        s1 = jnp.sum(We1_ref[:, sl] * ae1_ref[h:h + 1, :])
        att = col_softmax(dem * p1 + demt * q1 + mk * s1)
        th = jax.lax.dot_general(att, dem, (((0,), (0,)), ((), ())),
                                 preferred_element_type=f32)
        x1_blocks.append(jnp.maximum(th * W1h + b1_ref[:, sl], 0.0))
    x1 = jnp.concatenate(x1_blocks, axis=1)

    xs2 = jnp.dot(x1, W2_ref[:, :], preferred_element_type=f32)
    x2_blocks = []
    for h in range(H):
        sl = slice(h * C, (h + 1) * C)
        xsh = xs2[:, sl]
        s2 = jnp.sum(We2_ref[:, sl] * ae2_ref[h:h + 1, :])
        srcc = jax.lax.dot_general(xsh, as2_ref[h:h + 1, :],
                                   (((1,), (1,)), ((), ())),
                                   preferred_element_type=f32)
        dstr = jax.lax.dot_general(ad2_ref[h:h + 1, :], xsh,
                                   (((1,), (1,)), ((), ())),
                                   preferred_element_type=f32)
        att = col_softmax(srcc + dstr + mk * s2)
        oh = jax.lax.dot_general(att, xsh, (((0,), (0,)), ((), ())),
                                 preferred_element_type=f32)
        x2_blocks.append(jnp.maximum(oh + b2_ref[:, sl], 0.0))
    x2 = jnp.concatenate(x2_blocks, axis=1)

    A = jnp.dot(x2, wes_top_ref[:, :], preferred_element_type=f32)
    B = jnp.dot(x2, wes_bot_ref[:, :], preferred_element_type=f32) \
        + bes_ref[:, :]

    g1_parts = []
    c_parts = []
    CH = 64
    for jb in range(N // CH):
        w3c = w3_ref[jb * CH:(jb + 1) * CH, :, :]
        g1_parts.append(jnp.sum(w3c, axis=0))
        bc = B[jb * CH:(jb + 1) * CH, :]
        c_parts.append(jnp.sum(w3c * bc[:, :, None], axis=(0, 1),
                               keepdims=True)[0])
    G1 = g1_parts[0] + g1_parts[1] + g1_parts[2] + g1_parts[3]
    crow = c_parts[0] + c_parts[1] + c_parts[2] + c_parts[3]

    wd = idx_ref[0]
    cp = idx_ref[1]
    vh = idx_ref[2]
    wrow = wemb_ref[pl.ds(wd, 1), :]
    crow_e = cemb_ref[pl.ds(cp, 1), :]
    vrow = vemb_ref[pl.ds(vh, 1), :]
    wc = jax.lax.dot_general(wrow, wc1_w_ref[:, :], (((1,), (0,)), ((), ())),
                             preferred_element_type=f32)
    cc = jax.lax.dot_general(crow_e, wc1_c_ref[:, :], (((1,), (0,)), ((), ())),
                             preferred_element_type=f32)
    vc = jax.lax.dot_general(vrow, wc1_v_ref[:, :], (((1,), (0,)), ((), ())),
                             preferred_element_type=f32)

    iota = jax.lax.broadcasted_iota(jnp.int32, (64, N), 1)
    ind = jnp.max((stops_ref[:, :] == iota).astype(f32), axis=0,
                  keepdims=True)
    sfc = jax.lax.dot_general(ind, wc1_sf_ref[:, :], (((1,), (0,)), ((), ())),
                              preferred_element_type=f32)

    const_row = crow + wc + cc + vc + sfc + bc1_ref[:, :]

    hidden = jnp.dot(A, G1, preferred_element_type=f32)
    hidden = hidden + jnp.dot(dn, wc1_d_ref[:, :], preferred_element_type=f32)
    hidden = hidden + jnp.dot(mk, wc1_m_ref[:, :], preferred_element_type=f32)
    hidden = jnp.maximum(hidden + const_row, 0.0)
    out_ref[:, :] = jnp.dot(hidden, wc2_ref[:, :],
                            preferred_element_type=f32) + bc2_ref[:, :]


def kernel(dist, stops, weekday, vehicles, markov, demand, capacity, mask,
           W1, as1, ad1, We1, ae1, b1, W2, as2, ad2, We2, ae2, b2,
           Wes, bes, wemb, cemb, vemb, Wc1, bc1, Wc2, bc2):
    f32 = jnp.float32
    idx = jnp.stack([jnp.asarray(weekday, jnp.int32),
                     jnp.asarray(capacity, jnp.int32),
                     jnp.asarray(vehicles, jnp.int32)])

    demt = demand.reshape(1, N)
    stops2 = stops.astype(jnp.int32).reshape(64, 1)
    b1r = b1.reshape(1, HC)
    b2r = b2.reshape(1, HC)
    besr = bes.reshape(1, ER)
    bc1r = bc1.reshape(1, N)
    bc2r = bc2.reshape(1, N)
    wes_top = Wes[:HC, :]
    wes_bot = Wes[HC:, :]
    wemb_p = jnp.pad(wemb, ((0, 0), (0, 8 - FE)))
    cemb_p = jnp.pad(cemb, ((0, 0), (0, 8 - FE)))
    vemb_p = jnp.pad(vemb, ((0, 0), (0, 8 - FE)))
    w3 = Wc1[:N * ER, :].reshape(N, ER, N)
    wc1_d = Wc1[N * ER:N * ER + N, :]
    wc1_m = Wc1[N * ER + N:N * ER + 2 * N, :]
    o = N * ER + 2 * N
    wc1_w = jnp.pad(Wc1[o:o + FE, :], ((0, 8 - FE), (0, 0)))
    wc1_c = jnp.pad(Wc1[o + FE:o + 2 * FE, :], ((0, 8 - FE), (0, 0)))
    wc1_v = jnp.pad(Wc1[o + 2 * FE:o + 3 * FE, :], ((0, 8 - FE), (0, 0)))
    wc1_sf = Wc1[o + 3 * FE:, :]

    vmem = pl.BlockSpec(memory_space=pltpu.VMEM)
    smem = pl.BlockSpec(memory_space=pltpu.SMEM)
    return pl.pallas_call(
        _body,
        out_shape=jax.ShapeDtypeStruct((N, N), f32),
        in_specs=[smem] + [vmem] * 34,
        out_specs=vmem,
        compiler_params=pltpu.CompilerParams(
            vmem_limit_bytes=100 * 1024 * 1024),
    )(idx, dist, markov, mask, demand, demt, stops2,
      W1, as1, ad1, We1, ae1, b1r,
      W2, as2, ad2, We2, ae2, b2r,
      wes_top, wes_bot, besr, wemb_p, cemb_p, vemb_p,
      w3, wc1_d, wc1_m, wc1_w, wc1_c, wc1_v, wc1_sf, bc1r, Wc2, bc2r)


# speedup vs baseline: 539.9897x; 1.5143x over previous
import jax
import jax.numpy as jnp
from jax.experimental import pallas as pl
from jax.experimental.pallas import tpu as pltpu

N = 256
H = 8
C = 32
ER = 32
FE = 3
HC = H * C


def _body(idx_ref, dist_ref, markov_ref, mask_ref, dem_ref, demt_ref,
          stops_ref, W1_ref, as1_ref, ad1_ref, We1_ref, ae1_ref, b1_ref,
          W2_ref, as2_ref, ad2_ref, We2_ref, ae2_ref, b2_ref,
          wes_ref, bes_ref, wemb_ref, cemb_ref, vemb_ref,
          wc1_ref, bc1_ref, wc2_ref, bc2_ref, out_ref):
    f32 = jnp.float32

    dd = dist_ref[:, :]
    dn = (dd - jnp.min(dd)) / (jnp.max(dd) - jnp.min(dd))
    mk = markov_ref[:, :]
    valid = mask_ref[:, :] == 1
    dem = dem_ref[:, :]
    demt = demt_ref[:, :]

    def col_softmax(al):
        al = jnp.where(al >= 0.0, al, 0.2 * al)
        ex = jnp.where(valid, jnp.exp(al), 0.0)
        den = jnp.sum(ex, axis=0, keepdims=True)
        return ex * (1.0 / (den + 1e-16))

    x1_blocks = []
    for h in range(H):
        sl = slice(h * C, (h + 1) * C)
        W1h = W1_ref[:, sl]
        p1 = jnp.sum(W1h * as1_ref[h:h + 1, :])
        q1 = jnp.sum(W1h * ad1_ref[h:h + 1, :])
        s1 = jnp.sum(We1_ref[:, sl] * ae1_ref[h:h + 1, :])
        att = col_softmax(dem * p1 + demt * q1 + mk * s1)
        th = jax.lax.dot_general(att, dem, (((0,), (0,)), ((), ())),
                                 preferred_element_type=f32)
        x1_blocks.append(jnp.maximum(th * W1h + b1_ref[:, sl], 0.0))
    x1 = jnp.concatenate(x1_blocks, axis=1)

    xs2 = jnp.dot(x1, W2_ref[:, :], preferred_element_type=f32)
    x2_blocks = []
    for h in range(H):
        sl = slice(h * C, (h + 1) * C)
        xsh = xs2[:, sl]
        s2 = jnp.sum(We2_ref[:, sl] * ae2_ref[h:h + 1, :])
        srcc = jax.lax.dot_general(xsh, as2_ref[h:h + 1, :],
                                   (((1,), (1,)), ((), ())),
                                   preferred_element_type=f32)
        dstr = jax.lax.dot_general(ad2_ref[h:h + 1, :], xsh,
                                   (((1,), (1,)), ((), ())),
                                   preferred_element_type=f32)
        att = col_softmax(srcc + dstr + mk * s2)
        oh = jax.lax.dot_general(att, xsh, (((0,), (0,)), ((), ())),
                                 preferred_element_type=f32)
        x2_blocks.append(jnp.maximum(oh + b2_ref[:, sl], 0.0))
    x2 = jnp.concatenate(x2_blocks, axis=1)

    A = jnp.dot(x2, wes_ref[:HC, :], preferred_element_type=f32)
    B = jnp.dot(x2, wes_ref[HC:, :], preferred_element_type=f32) \
        + bes_ref[:, :]

    g1_parts = []
    c_parts = []
    CH = 64
    for jb in range(N // CH):
        w3c = wc1_ref[jb * CH * ER:(jb + 1) * CH * ER, :].reshape(CH, ER, N)
        g1_parts.append(jnp.sum(w3c, axis=0))
        bc = B[jb * CH:(jb + 1) * CH, :]
        c_parts.append(jnp.sum(w3c * bc[:, :, None], axis=(0, 1),
                               keepdims=True)[0])
    G1 = g1_parts[0] + g1_parts[1] + g1_parts[2] + g1_parts[3]
    crow = c_parts[0] + c_parts[1] + c_parts[2] + c_parts[3]

    o = N * ER + 2 * N
    wd = idx_ref[0]
    cp = idx_ref[1]
    vh = idx_ref[2]
    wrow = wemb_ref[pl.ds(wd, 1), :]
    crow_e = cemb_ref[pl.ds(cp, 1), :]
    vrow = vemb_ref[pl.ds(vh, 1), :]
    wc = jax.lax.dot_general(wrow, wc1_ref[o:o + 8, :],
                             (((1,), (0,)), ((), ())),
                             preferred_element_type=f32)
    cc = jax.lax.dot_general(crow_e, wc1_ref[o + FE:o + FE + 8, :],
                             (((1,), (0,)), ((), ())),
                             preferred_element_type=f32)
    vc = jax.lax.dot_general(vrow, wc1_ref[o + 2 * FE:o + 2 * FE + 8, :],
                             (((1,), (0,)), ((), ())),
                             preferred_element_type=f32)

    iota = jax.lax.broadcasted_iota(jnp.int32, (64, N), 1)
    ind = jnp.max((stops_ref[:, :] == iota).astype(f32), axis=0,
                  keepdims=True)
    sfc = jax.lax.dot_general(ind, wc1_ref[o + 3 * FE:, :],
                              (((1,), (0,)), ((), ())),
                              preferred_element_type=f32)

    const_row = crow + wc + cc + vc + sfc + bc1_ref[:, :]

    dm = jnp.concatenate([dn, mk], axis=1)
    hidden = jnp.dot(A, G1, preferred_element_type=f32)
    hidden = hidden + jnp.dot(dm, wc1_ref[N * ER:N * ER + 2 * N, :],
                              preferred_element_type=f32)
    hidden = jnp.maximum(hidden + const_row, 0.0)
    out_ref[:, :] = jnp.dot(hidden, wc2_ref[:, :],
                            preferred_element_type=f32) + bc2_ref[:, :]


def kernel(dist, stops, weekday, vehicles, markov, demand, capacity, mask,
           W1, as1, ad1, We1, ae1, b1, W2, as2, ad2, We2, ae2, b2,
           Wes, bes, wemb, cemb, vemb, Wc1, bc1, Wc2, bc2):
    f32 = jnp.float32
    idx = jnp.stack([jnp.asarray(weekday, jnp.int32),
                     jnp.asarray(capacity, jnp.int32),
                     jnp.asarray(vehicles, jnp.int32)])

    demt = demand.reshape(1, N)
    stops2 = stops.astype(jnp.int32).reshape(64, 1)
    b1r = b1.reshape(1, HC)
    b2r = b2.reshape(1, HC)
    besr = bes.reshape(1, ER)
    bc1r = bc1.reshape(1, N)
    bc2r = bc2.reshape(1, N)
    wemb_p = jnp.pad(wemb, ((0, 0), (0, 8 - FE)))
    cemb_p = jnp.pad(cemb, ((0, 0), (0, 8 - FE)))
    vemb_p = jnp.pad(vemb, ((0, 0), (0, 8 - FE)))

    vmem = pl.BlockSpec(memory_space=pltpu.VMEM)
    smem = pl.BlockSpec(memory_space=pltpu.SMEM)
    return pl.pallas_call(
        _body,
        out_shape=jax.ShapeDtypeStruct((N, N), f32),
        in_specs=[smem] + [vmem] * 27,
        out_specs=vmem,
        compiler_params=pltpu.CompilerParams(
            vmem_limit_bytes=100 * 1024 * 1024),
    )(idx, dist, markov, mask, demand, demt, stops2,
      W1, as1, ad1, We1, ae1, b1r,
      W2, as2, ad2, We2, ae2, b2r,
      Wes, besr, wemb_p, cemb_p, vemb_p,
      Wc1, bc1r, Wc2, bc2r)
